# baseline (speedup 1.0000x reference)
import jax
import jax.numpy as jnp
from jax import lax
from jax.experimental import pallas as pl
from jax.experimental.pallas import tpu as pltpu
from jax.experimental.pallas import tpu_sc as plsc

NC = 2
NS = 16
NW = NC * NS

V = 100000
B_CTX = 4096
B_LIST = 50
DIM = 64
DPW = DIM // NW
NLANE = 16


def _gather_all(vrow, idsbuf, outbuf, n):
    @pl.loop(0, n // NLANE, unroll=8)
    def _(i):
        idx = idsbuf[pl.ds(i * NLANE, NLANE)]
        outbuf[pl.ds(i * NLANE, NLANE)] = plsc.load_gather(vrow, [idx])


def _sc_body(ctxT, exT, ctx_ids, ex_idsT, ctx_outT, ex_outT,
             vrow, idsbuf, outbuf, rsem, isem, osem):
    wid = lax.axis_index("s") * NC + lax.axis_index("c")

    for k in range(DPW):
        dim = wid * DPW + k

        pltpu.async_copy(ctxT.at[dim], vrow, rsem).wait()
        pltpu.sync_copy(ctx_ids, idsbuf)
        _gather_all(vrow, idsbuf, outbuf, B_CTX)
        pltpu.sync_copy(outbuf, ctx_outT.at[dim])

        pltpu.async_copy(exT.at[dim], vrow, rsem).wait()

        @pl.loop(0, B_LIST)
        def _(l):
            pltpu.async_copy(ex_idsT.at[l], idsbuf, isem).wait()
            _gather_all(vrow, idsbuf, outbuf, B_CTX)
            pltpu.async_copy(outbuf, ex_outT.at[l, dim], osem).wait()


@jax.jit
def _encode(ctxT, exT, ctx_ids, ex_idsT):
    mesh = plsc.VectorSubcoreMesh(core_axis_name="c", subcore_axis_name="s")
    ctx_outT, ex_outT = pl.kernel(
        _sc_body,
        out_type=(
            jax.ShapeDtypeStruct((DIM, B_CTX), jnp.float32),
            jax.ShapeDtypeStruct((B_LIST, DIM, B_CTX), jnp.float32),
        ),
        mesh=mesh,
        compiler_params=pltpu.CompilerParams(needs_layout_passes=False),
        scratch_types=[
            pltpu.VMEM((V,), jnp.float32),
            pltpu.VMEM((B_CTX,), jnp.int32),
            pltpu.VMEM((B_CTX,), jnp.float32),
            pltpu.SemaphoreType.DMA,
            pltpu.SemaphoreType.DMA,
            pltpu.SemaphoreType.DMA,
        ],
    )(ctxT, exT, ctx_ids, ex_idsT)
    return ctx_outT, ex_outT


def kernel(context_table, example_table, context_ids, example_ids):
    ctxT = context_table.T
    exT = example_table.T
    ctx_ids = jnp.asarray(context_ids, jnp.int32)
    ex_idsT = jnp.asarray(example_ids, jnp.int32).T
    ctx_outT, ex_outT = _encode(ctxT, exT, ctx_ids, ex_idsT)
    return ctx_outT.T, jnp.transpose(ex_outT, (2, 0, 1))


# speedup vs baseline: 1.3721x; 1.3721x over previous
import jax
import jax.numpy as jnp
from jax import lax
from jax.experimental import pallas as pl
from jax.experimental.pallas import tpu as pltpu
from jax.experimental.pallas import tpu_sc as plsc

NC = 2
NS = 16
NW = NC * NS

V = 100000
B_CTX = 4096
B_LIST = 50
DIM = 64
DPW = DIM // NW
NLANE = 16


def _gather_all(vrow, idsbuf, outbuf, n):
    @pl.loop(0, n // NLANE, unroll=8)
    def _(i):
        idx = idsbuf[pl.ds(i * NLANE, NLANE)]
        outbuf[pl.ds(i * NLANE, NLANE)] = plsc.load_gather(vrow, [idx])


def _sc_body(ctxT, exT, ctx_ids, ex_idsT, ctx_outT, ex_outT,
             vrow, cidsbuf, coutbuf, ids_a, ids_b, out_a, out_b, rsem, csem,
             isems, osems):
    wid = lax.axis_index("s") * NC + lax.axis_index("c")

    for k in range(DPW):
        dim = wid * DPW + k

        pltpu.async_copy(ctxT.at[dim], vrow, rsem).wait()
        if k == 0:
            pltpu.sync_copy(ctx_ids, cidsbuf)
        _gather_all(vrow, cidsbuf, coutbuf, B_CTX)
        ctx_store = pltpu.async_copy(coutbuf, ctx_outT.at[dim], csem)

        pltpu.async_copy(exT.at[dim], vrow, rsem).wait()

        idsb = (ids_a, ids_b)
        outb = (out_a, out_b)

        def ids_start(c, b):
            pltpu.async_copy(ex_idsT.at[c], idsb[b], isems.at[b])

        def ids_wait(c, b):
            pltpu.make_async_copy(
                ex_idsT.at[c], idsb[b], isems.at[b]).wait()

        def store_start(c, b):
            pltpu.async_copy(outb[b], ex_outT.at[c, dim], osems.at[b])

        def store_wait(c, b):
            pltpu.make_async_copy(
                outb[b], ex_outT.at[c, dim], osems.at[b]).wait()

        ids_start(0, 0)
        ids_start(1, 1)
        for c in (0, 1):
            ids_wait(c, c)
            _gather_all(vrow, idsb[c], outb[c], B_CTX)
            store_start(c, c)
            ids_start(c + 2, c)

        @pl.loop(2, B_LIST - 2, step=2)
        def _(j):
            for b in range(2):
                c = j + b
                ids_wait(c, b)
                store_wait(c - 2, b)
                _gather_all(vrow, idsb[b], outb[b], B_CTX)
                store_start(c, b)
                ids_start(c + 2, b)

        for c in (B_LIST - 2, B_LIST - 1):
            b = c % 2
            ids_wait(c, b)
            store_wait(c - 2, b)
            _gather_all(vrow, idsb[b], outb[b], B_CTX)
            store_start(c, b)
        for c in (B_LIST - 2, B_LIST - 1):
            store_wait(c, c % 2)
        ctx_store.wait()


@jax.jit
def _encode(ctxT, exT, ctx_ids, ex_idsT):
    mesh = plsc.VectorSubcoreMesh(core_axis_name="c", subcore_axis_name="s")
    ctx_outT, ex_outT = pl.kernel(
        _sc_body,
        out_type=(
            jax.ShapeDtypeStruct((DIM, B_CTX), jnp.float32),
            jax.ShapeDtypeStruct((B_LIST, DIM, B_CTX), jnp.float32),
        ),
        mesh=mesh,
        compiler_params=pltpu.CompilerParams(needs_layout_passes=False),
        scratch_types=[
            pltpu.VMEM((V,), jnp.float32),
            pltpu.VMEM((B_CTX,), jnp.int32),
            pltpu.VMEM((B_CTX,), jnp.float32),
            pltpu.VMEM((B_CTX,), jnp.int32),
            pltpu.VMEM((B_CTX,), jnp.int32),
            pltpu.VMEM((B_CTX,), jnp.float32),
            pltpu.VMEM((B_CTX,), jnp.float32),
            pltpu.SemaphoreType.DMA,
            pltpu.SemaphoreType.DMA,
            pltpu.SemaphoreType.DMA((2,)),
            pltpu.SemaphoreType.DMA((2,)),
        ],
    )(ctxT, exT, ctx_ids, ex_idsT)
    return ctx_outT, ex_outT


def kernel(context_table, example_table, context_ids, example_ids):
    ctxT = context_table.T
    exT = example_table.T
    ctx_ids = jnp.asarray(context_ids, jnp.int32)
    ex_idsT = jnp.asarray(example_ids, jnp.int32).T
    ctx_outT, ex_outT = _encode(ctxT, exT, ctx_ids, ex_idsT)
    return ctx_outT.T, jnp.transpose(ex_outT, (2, 0, 1))


# speedup vs baseline: 2.9820x; 2.1733x over previous
import jax
import jax.numpy as jnp
from jax import lax
from jax.experimental import pallas as pl
from jax.experimental.pallas import tpu as pltpu
from jax.experimental.pallas import tpu_sc as plsc

NC = 2
NS = 16
NW = NC * NS

V = 100000
B_CTX = 4096
B_LIST = 50
DIM = 64
DPW = DIM // NW
NLANE = 16


def _gather_all(vrow, idsbuf, outbuf, n):
    U = 8
    @pl.loop(0, n // (NLANE * U))
    def _(i):
        base = i * (NLANE * U)
        idxs = [idsbuf[pl.ds(base + u * NLANE, NLANE)] for u in range(U)]
        vals = [plsc.load_gather(vrow, [ix]) for ix in idxs]
        for u in range(U):
            outbuf[pl.ds(base + u * NLANE, NLANE)] = vals[u]


def _sc_body(ctxT, exT, ctx_ids, ex_idsT, ctx_outT, ex_outT,
             vrow, cidsbuf, coutbuf, ids_a, ids_b, out_a, out_b, rsem, csem,
             isems, osems):
    wid = lax.axis_index("s") * NC + lax.axis_index("c")

    for k in range(DPW):
        dim = wid * DPW + k

        pltpu.async_copy(ctxT.at[dim], vrow, rsem).wait()
        if k == 0:
            pltpu.sync_copy(ctx_ids, cidsbuf)
        _gather_all(vrow, cidsbuf, coutbuf, B_CTX)
        ctx_store = pltpu.async_copy(coutbuf, ctx_outT.at[dim], csem)

        pltpu.async_copy(exT.at[dim], vrow, rsem).wait()

        idsb = (ids_a, ids_b)
        outb = (out_a, out_b)

        def ids_start(c, b):
            pltpu.async_copy(ex_idsT.at[c], idsb[b], isems.at[b])

        def ids_wait(c, b):
            pltpu.make_async_copy(
                ex_idsT.at[c], idsb[b], isems.at[b]).wait()

        def store_start(c, b):
            pltpu.async_copy(outb[b], ex_outT.at[c, dim], osems.at[b])

        def store_wait(c, b):
            pltpu.make_async_copy(
                outb[b], ex_outT.at[c, dim], osems.at[b]).wait()

        ids_start(0, 0)
        ids_start(1, 1)
        for c in (0, 1):
            ids_wait(c, c)
            _gather_all(vrow, idsb[c], outb[c], B_CTX)
            store_start(c, c)
            ids_start(c + 2, c)

        @pl.loop(2, B_LIST - 2, step=2)
        def _(j):
            for b in range(2):
                c = j + b
                ids_wait(c, b)
                store_wait(c - 2, b)
                _gather_all(vrow, idsb[b], outb[b], B_CTX)
                store_start(c, b)
                ids_start(c + 2, b)

        for c in (B_LIST - 2, B_LIST - 1):
            b = c % 2
            ids_wait(c, b)
            store_wait(c - 2, b)
            _gather_all(vrow, idsb[b], outb[b], B_CTX)
            store_start(c, b)
        for c in (B_LIST - 2, B_LIST - 1):
            store_wait(c, c % 2)
        ctx_store.wait()


@jax.jit
def _encode(ctxT, exT, ctx_ids, ex_idsT):
    mesh = plsc.VectorSubcoreMesh(core_axis_name="c", subcore_axis_name="s")
    ctx_outT, ex_outT = pl.kernel(
        _sc_body,
        out_type=(
            jax.ShapeDtypeStruct((DIM, B_CTX), jnp.float32),
            jax.ShapeDtypeStruct((B_LIST, DIM, B_CTX), jnp.float32),
        ),
        mesh=mesh,
        compiler_params=pltpu.CompilerParams(needs_layout_passes=False),
        scratch_types=[
            pltpu.VMEM((V,), jnp.float32),
            pltpu.VMEM((B_CTX,), jnp.int32),
            pltpu.VMEM((B_CTX,), jnp.float32),
            pltpu.VMEM((B_CTX,), jnp.int32),
            pltpu.VMEM((B_CTX,), jnp.int32),
            pltpu.VMEM((B_CTX,), jnp.float32),
            pltpu.VMEM((B_CTX,), jnp.float32),
            pltpu.SemaphoreType.DMA,
            pltpu.SemaphoreType.DMA,
            pltpu.SemaphoreType.DMA((2,)),
            pltpu.SemaphoreType.DMA((2,)),
        ],
    )(ctxT, exT, ctx_ids, ex_idsT)
    return ctx_outT, ex_outT


def kernel(context_table, example_table, context_ids, example_ids):
    ctxT = context_table.T
    exT = example_table.T
    ctx_ids = jnp.asarray(context_ids, jnp.int32)
    ex_idsT = jnp.asarray(example_ids, jnp.int32).T
    ctx_outT, ex_outT = _encode(ctxT, exT, ctx_ids, ex_idsT)
    return ctx_outT.T, jnp.transpose(ex_outT, (2, 0, 1))
